# Initial kernel scaffold; baseline (speedup 1.0000x reference)
import jax
import jax.numpy as jnp
from jax.experimental import pallas as pl


def kernel(batch_feat, abs_proxy_label, camid, pseudo_cluster_label, memory, epoch, k, inter_loss_epoch):
    raise NotImplementedError("write your pallas kernel here")


# speedup vs baseline: 6.7486x; 6.7486x over previous
import functools

import jax
import jax.numpy as jnp
from jax.experimental import pallas as pl

N_PROXIES = 8192
N_CAMS = 8
PPC = N_PROXIES // N_CAMS
TEMP = 0.07
B = 1024
D = 256


def _percam_kernel(feat_ref, mem_ref, cam_ref, tgt_ref, out_ref):
    c = pl.program_id(0)
    x = feat_ref[...]
    w = mem_ref[...]
    sim = jax.lax.dot_general(
        x, w, (((1,), (1,)), ((), ())), preferred_element_type=jnp.float32
    ) * (1.0 / TEMP)
    m = jnp.max(sim, axis=1, keepdims=True)
    lse = jnp.log(jnp.sum(jnp.exp(sim - m), axis=1)) + m[:, 0]
    tgt = tgt_ref[0, 0, :]
    cols = jax.lax.broadcasted_iota(jnp.int32, (B, PPC), 1)
    tlogit = jnp.sum(jnp.where(cols == tgt[:, None], sim, 0.0), axis=1)
    maskf = (cam_ref[0, 0, :] == c).astype(jnp.float32)
    row_loss = (lse - tlogit) * maskf
    s = jnp.sum(row_loss)
    cnt = jnp.sum(maskf)
    lane = jax.lax.broadcasted_iota(jnp.int32, (1, 1, 128), 2)
    out_ref[...] = jnp.where(lane == 0, s, jnp.where(lane == 1, cnt, 0.0))


def kernel(batch_feat, abs_proxy_label, camid, pseudo_cluster_label, memory,
           epoch, k, inter_loss_epoch):
    cam3 = camid.reshape(1, 1, B).astype(jnp.int32)
    tgt3 = (abs_proxy_label % PPC).reshape(1, 1, B).astype(jnp.int32)
    out = pl.pallas_call(
        _percam_kernel,
        grid=(N_CAMS,),
        in_specs=[
            pl.BlockSpec((B, D), lambda c: (0, 0)),
            pl.BlockSpec((PPC, D), lambda c: (c, 0)),
            pl.BlockSpec((1, 1, B), lambda c: (0, 0, 0)),
            pl.BlockSpec((1, 1, B), lambda c: (0, 0, 0)),
        ],
        out_specs=pl.BlockSpec((1, 1, 128), lambda c: (c, 0, 0)),
        out_shape=jax.ShapeDtypeStruct((N_CAMS, 1, 128), jnp.float32),
    )(batch_feat, memory, cam3, tgt3)
    sums = out[:, 0, 0]
    cnts = out[:, 0, 1]
    return jnp.sum(jnp.where(cnts > 0, sums / jnp.maximum(cnts, 1.0), 0.0))
